# baseline (speedup 1.0000x reference)
import functools

import jax
import jax.numpy as jnp
from jax import lax
from jax.experimental import pallas as pl
from jax.experimental.pallas import tpu as pltpu
from jax.experimental.pallas import tpu_sc as plsc

N_NODES = 10000
N_EDGES = 160000
D_FEAT = 256

_NUM_CORES = 2
_NUM_SUBCORES = 16
_NW = _NUM_CORES * _NUM_SUBCORES
_B_PER_W = N_EDGES // _NW
_CHUNK = 40
_NB = 8
_GRP = _NB * _CHUNK
_NGRP = _B_PER_W // _GRP
_TAIL = _B_PER_W - _NGRP * _GRP

_mesh = plsc.VectorSubcoreMesh(core_axis_name="c", subcore_axis_name="s")


@functools.partial(
    pl.kernel,
    mesh=_mesh,
    out_type=jax.ShapeDtypeStruct((N_EDGES, D_FEAT), jnp.float32),
    scratch_types=(
        [pltpu.VMEM((_B_PER_W,), jnp.int32)]
        + [pltpu.VMEM((_CHUNK, D_FEAT), jnp.float32) for _ in range(_NB)]
        + [pltpu.SemaphoreType.DMA for _ in range(2 * _NB)]
    ),
)
def _gather_rows(idx_hbm, x_hbm, out_hbm, idx_v, *bufs_and_sems):
    rows = bufs_and_sems[:_NB]
    sem_g = bufs_and_sems[_NB:2 * _NB]
    sem_s = bufs_and_sems[2 * _NB:]
    wid = lax.axis_index("s") * _NUM_CORES + lax.axis_index("c")
    base = wid * _B_PER_W
    pltpu.sync_copy(idx_hbm.at[pl.ds(base, _B_PER_W)], idx_v)

    def body(s, carry):
        goff = s * _GRP
        for b in range(_NB):
            @pl.when(s > 0)
            def _drain(b=b):
                pltpu.make_async_copy(
                    rows[b], out_hbm.at[pl.ds(base, _CHUNK)], sem_s[b]
                ).wait()
            pltpu.async_copy(
                x_hbm.at[idx_v.at[pl.ds(goff + b * _CHUNK, _CHUNK)]],
                rows[b], sem_g[b],
            )
        for b in range(_NB):
            off = goff + b * _CHUNK
            pltpu.make_async_copy(
                x_hbm.at[idx_v.at[pl.ds(off, _CHUNK)]], rows[b], sem_g[b]
            ).wait()
            pltpu.async_copy(rows[b], out_hbm.at[pl.ds(base + off, _CHUNK)],
                             sem_s[b])
        return carry

    lax.fori_loop(0, _NGRP, body, 0)
    for b in range(_NB):
        pltpu.make_async_copy(
            rows[b], out_hbm.at[pl.ds(base, _CHUNK)], sem_s[b]
        ).wait()
    toff = _NGRP * _GRP
    rem = _TAIL
    while rem:
        c = min(128, rem)
        pltpu.async_copy(
            x_hbm.at[idx_v.at[pl.ds(toff, c)]],
            rows[0].at[pl.ds(0, c)], sem_g[0],
        ).wait()
        pltpu.sync_copy(rows[0].at[pl.ds(0, c)],
                        out_hbm.at[pl.ds(base + toff, c)])
        toff += c
        rem -= c


def kernel(edge_index, x):
    idx = edge_index[0].astype(jnp.int32)
    return _gather_rows(idx, x)


# speedup vs baseline: 1.0018x; 1.0018x over previous
import functools

import jax
import jax.numpy as jnp
from jax import lax
from jax.experimental import pallas as pl
from jax.experimental.pallas import tpu as pltpu
from jax.experimental.pallas import tpu_sc as plsc

N_NODES = 10000
N_EDGES = 160000
D_FEAT = 256

_NUM_CORES = 2
_NUM_SUBCORES = 16
_NW = _NUM_CORES * _NUM_SUBCORES
_B_PER_W = N_EDGES // _NW
_CHUNK = 64
_NB = 5
_GRP = _NB * _CHUNK
_NGRP = _B_PER_W // _GRP
_TAIL = _B_PER_W - _NGRP * _GRP

_mesh = plsc.VectorSubcoreMesh(core_axis_name="c", subcore_axis_name="s")


@functools.partial(
    pl.kernel,
    mesh=_mesh,
    out_type=jax.ShapeDtypeStruct((N_EDGES, D_FEAT), jnp.float32),
    scratch_types=(
        [pltpu.VMEM((_B_PER_W,), jnp.int32)]
        + [pltpu.VMEM((_CHUNK, D_FEAT), jnp.float32) for _ in range(_NB)]
        + [pltpu.SemaphoreType.DMA for _ in range(2 * _NB)]
    ),
)
def _gather_rows(idx_hbm, x_hbm, out_hbm, idx_v, *bufs_and_sems):
    rows = bufs_and_sems[:_NB]
    sem_g = bufs_and_sems[_NB:2 * _NB]
    sem_s = bufs_and_sems[2 * _NB:]
    wid = lax.axis_index("s") * _NUM_CORES + lax.axis_index("c")
    base = wid * _B_PER_W
    pltpu.sync_copy(idx_hbm.at[pl.ds(base, _B_PER_W)], idx_v)

    def body(s, carry):
        goff = s * _GRP
        for b in range(_NB):
            @pl.when(s > 0)
            def _drain(b=b):
                pltpu.make_async_copy(
                    rows[b], out_hbm.at[pl.ds(base, _CHUNK)], sem_s[b]
                ).wait()
            pltpu.async_copy(
                x_hbm.at[idx_v.at[pl.ds(goff + b * _CHUNK, _CHUNK)]],
                rows[b], sem_g[b],
            )
        for b in range(_NB):
            off = goff + b * _CHUNK
            pltpu.make_async_copy(
                x_hbm.at[idx_v.at[pl.ds(off, _CHUNK)]], rows[b], sem_g[b]
            ).wait()
            pltpu.async_copy(rows[b], out_hbm.at[pl.ds(base + off, _CHUNK)],
                             sem_s[b])
        return carry

    lax.fori_loop(0, _NGRP, body, 0)
    for b in range(_NB):
        pltpu.make_async_copy(
            rows[b], out_hbm.at[pl.ds(base, _CHUNK)], sem_s[b]
        ).wait()
    toff = _NGRP * _GRP
    rem = _TAIL
    while rem:
        c = min(128, rem)
        pltpu.async_copy(
            x_hbm.at[idx_v.at[pl.ds(toff, c)]],
            rows[0].at[pl.ds(0, c)], sem_g[0],
        ).wait()
        pltpu.sync_copy(rows[0].at[pl.ds(0, c)],
                        out_hbm.at[pl.ds(base + toff, c)])
        toff += c
        rem -= c


def kernel(edge_index, x):
    idx = edge_index[0].astype(jnp.int32)
    return _gather_rows(idx, x)


# speedup vs baseline: 1.0216x; 1.0197x over previous
import functools

import jax
import jax.numpy as jnp
from jax import lax
from jax.experimental import pallas as pl
from jax.experimental.pallas import tpu as pltpu
from jax.experimental.pallas import tpu_sc as plsc

N_NODES = 10000
N_EDGES = 160000
D_FEAT = 256

_NUM_CORES = 2
_NUM_SUBCORES = 16
_NW = _NUM_CORES * _NUM_SUBCORES
_B_PER_W = N_EDGES // _NW
_CHUNK = 40
_NB = 5
_GRP = _NB * _CHUNK
_NGRP = _B_PER_W // _GRP
_TAIL = _B_PER_W - _NGRP * _GRP

_mesh = plsc.VectorSubcoreMesh(core_axis_name="c", subcore_axis_name="s")


@functools.partial(
    pl.kernel,
    mesh=_mesh,
    out_type=jax.ShapeDtypeStruct((N_EDGES, D_FEAT), jnp.float32),
    scratch_types=(
        [pltpu.VMEM((_B_PER_W,), jnp.int32)]
        + [pltpu.VMEM((_CHUNK, D_FEAT), jnp.float32) for _ in range(_NB)]
        + [pltpu.SemaphoreType.DMA for _ in range(2 * _NB)]
    ),
)
def _gather_rows(idx_hbm, x_hbm, out_hbm, idx_v, *bufs_and_sems):
    rows = bufs_and_sems[:_NB]
    sem_g = bufs_and_sems[_NB:2 * _NB]
    sem_s = bufs_and_sems[2 * _NB:]
    wid = lax.axis_index("s") * _NUM_CORES + lax.axis_index("c")
    base = wid * _B_PER_W
    pltpu.sync_copy(idx_hbm.at[pl.ds(base, _B_PER_W)], idx_v)

    def body(s, carry):
        goff = s * _GRP
        for b in range(_NB):
            @pl.when(s > 0)
            def _drain(b=b):
                pltpu.make_async_copy(
                    rows[b], out_hbm.at[pl.ds(base, _CHUNK)], sem_s[b]
                ).wait()
            pltpu.async_copy(
                x_hbm.at[idx_v.at[pl.ds(goff + b * _CHUNK, _CHUNK)]],
                rows[b], sem_g[b],
            )
        for b in range(_NB):
            off = goff + b * _CHUNK
            pltpu.make_async_copy(
                x_hbm.at[idx_v.at[pl.ds(off, _CHUNK)]], rows[b], sem_g[b]
            ).wait()
            pltpu.async_copy(rows[b], out_hbm.at[pl.ds(base + off, _CHUNK)],
                             sem_s[b])
        return carry

    lax.fori_loop(0, _NGRP, body, 0)
    for b in range(_NB):
        pltpu.make_async_copy(
            rows[b], out_hbm.at[pl.ds(base, _CHUNK)], sem_s[b]
        ).wait()
    toff = _NGRP * _GRP
    rem = _TAIL
    while rem:
        c = min(128, rem)
        pltpu.async_copy(
            x_hbm.at[idx_v.at[pl.ds(toff, c)]],
            rows[0].at[pl.ds(0, c)], sem_g[0],
        ).wait()
        pltpu.sync_copy(rows[0].at[pl.ds(0, c)],
                        out_hbm.at[pl.ds(base + toff, c)])
        toff += c
        rem -= c


def kernel(edge_index, x):
    idx = edge_index[0].astype(jnp.int32)
    return _gather_rows(idx, x)
